# baseline (speedup 1.0000x reference)
import functools
import jax
import jax.numpy as jnp
from jax import lax
from jax.experimental import pallas as pl
from jax.experimental.pallas import tpu as pltpu, tpu_sc as plsc

C_ROWS = 8


def kernel(x, emb_weight):
    B, T, D = x.shape
    NW = 32
    rows_w = T // NW
    n_chunks = rows_w // C_ROWS
    chunk_elems = C_ROWS * D
    n_vec = chunk_elems // 16

    x_flat = x.reshape(B * T * D)
    emb_flat = emb_weight[:T].reshape(T * D)
    mesh = plsc.VectorSubcoreMesh(core_axis_name="c", subcore_axis_name="s")

    @functools.partial(
        pl.kernel,
        mesh=mesh,
        out_type=jax.ShapeDtypeStruct((B * T * D,), jnp.float32),
        scratch_types=[
            pltpu.VMEM((chunk_elems,), jnp.float32),
            pltpu.VMEM((chunk_elems,), jnp.float32),
        ],
    )
    def k(x_hbm, emb_hbm, out_hbm, emb_v, x_v):
        wid = lax.axis_index("s") * 2 + lax.axis_index("c")
        t_base = wid * rows_w

        def chunk_body(ci, _):
            e_off = (t_base + ci * C_ROWS) * D
            pltpu.sync_copy(emb_hbm.at[pl.ds(e_off, chunk_elems)], emb_v)

            def batch_body(b, _):
                x_off = (b * T + t_base + ci * C_ROWS) * D
                pltpu.sync_copy(x_hbm.at[pl.ds(x_off, chunk_elems)], x_v)

                def add_body(i, _):
                    sl = pl.ds(i * 16, 16)
                    x_v[sl] = x_v[sl] + emb_v[sl]
                    return 0

                lax.fori_loop(0, n_vec, add_body, 0)
                pltpu.sync_copy(x_v, out_hbm.at[pl.ds(x_off, chunk_elems)])
                return 0

            lax.fori_loop(0, B, batch_body, 0)
            return 0

        lax.fori_loop(0, n_chunks, chunk_body, 0)

    out = k(x_flat, emb_flat)
    return out.reshape(B, T, D)


# speedup vs baseline: 1.1383x; 1.1383x over previous
import functools
import jax
import jax.numpy as jnp
from jax import lax
from jax.experimental import pallas as pl
from jax.experimental.pallas import tpu as pltpu, tpu_sc as plsc

C_ROWS = 8
UNROLL = 8


def kernel(x, emb_weight):
    B, T, D = x.shape
    NW = 32
    rows_w = T // NW
    n_chunks = rows_w // C_ROWS
    CE = C_ROWS * D
    n_vec = CE // 16

    x_flat = x.reshape(B * T * D)
    emb_flat = emb_weight[:T].reshape(T * D)
    mesh = plsc.VectorSubcoreMesh(core_axis_name="c", subcore_axis_name="s")

    @functools.partial(
        pl.kernel,
        mesh=mesh,
        out_type=jax.ShapeDtypeStruct((B * T * D,), jnp.float32),
        scratch_types=[
            pltpu.VMEM((2, B, CE), jnp.float32),
            pltpu.VMEM((2, CE), jnp.float32),
            pltpu.SemaphoreType.DMA((2, B)),
            pltpu.SemaphoreType.DMA((2,)),
            pltpu.SemaphoreType.DMA((2, B)),
        ],
    )
    def k(x_hbm, emb_hbm, out_hbm, xb, eb, sx, se, st):
        wid = lax.axis_index("s") * 2 + lax.axis_index("c")
        t0 = wid * rows_w * D

        handles = {}

        def load_chunk(ci):
            p = ci % 2
            e_off = t0 + ci * CE
            handles[("e", ci)] = pltpu.async_copy(
                emb_hbm.at[pl.ds(e_off, CE)], eb.at[p], se.at[p])
            for b in range(B):
                x_off = b * (T * D) + e_off
                handles[("x", ci, b)] = pltpu.async_copy(
                    x_hbm.at[pl.ds(x_off, CE)], xb.at[p, b], sx.at[p, b])

        load_chunk(0)
        for ci in range(n_chunks):
            p = ci % 2
            if ci + 1 < n_chunks:
                if ci >= 1:
                    for b in range(B):
                        handles[("s", ci - 1, b)].wait()
                load_chunk(ci + 1)
            handles[("e", ci)].wait()
            for b in range(B):
                handles[("x", ci, b)].wait()

                def add_body(i, _):
                    base = i * (16 * UNROLL)
                    for u in range(UNROLL):
                        sl = pl.ds(base + u * 16, 16)
                        xb[p, b, sl] = xb[p, b, sl] + eb[p, sl]
                    return 0

                lax.fori_loop(0, n_vec // UNROLL, add_body, 0)
                x_off = b * (T * D) + t0 + ci * CE
                handles[("s", ci, b)] = pltpu.async_copy(
                    xb.at[p, b], out_hbm.at[pl.ds(x_off, CE)], st.at[p, b])
        for b in range(B):
            handles[("s", n_chunks - 2, b)].wait()
            handles[("s", n_chunks - 1, b)].wait()

    out = k(x_flat, emb_flat)
    return out.reshape(B, T, D)


# speedup vs baseline: 1.6600x; 1.4583x over previous
import functools
import jax
import jax.numpy as jnp
from jax import lax
from jax.experimental import pallas as pl
from jax.experimental.pallas import tpu as pltpu, tpu_sc as plsc

C_ROWS = 8
UNROLL = 8


def kernel(x, emb_weight):
    B, T, D = x.shape
    NW = 32
    rows_w = T // NW
    n_chunks = rows_w // C_ROWS
    CE = C_ROWS * D
    n_vec = CE // 16

    x_flat = x.reshape(B * T * D)
    emb_flat = emb_weight[:T].reshape(T * D)
    mesh = plsc.VectorSubcoreMesh(core_axis_name="c", subcore_axis_name="s")

    @functools.partial(
        pl.kernel,
        mesh=mesh,
        out_type=jax.ShapeDtypeStruct((B * T * D,), jnp.float32),
        scratch_types=[
            pltpu.VMEM((2, B, CE), jnp.float32),
            pltpu.VMEM((2, CE), jnp.float32),
            pltpu.SemaphoreType.DMA((2, B)),
            pltpu.SemaphoreType.DMA((2,)),
            pltpu.SemaphoreType.DMA((2, B)),
        ],
    )
    def k(x_hbm, emb_hbm, out_hbm, xb, eb, sx, se, st):
        wid = lax.axis_index("s") * 2 + lax.axis_index("c")
        t0 = wid * rows_w * D

        handles = {}

        def load_chunk(ci):
            p = ci % 2
            e_off = t0 + ci * CE
            handles[("e", ci)] = pltpu.async_copy(
                emb_hbm.at[pl.ds(e_off, CE)], eb.at[p], se.at[p])
            for b in range(B):
                x_off = b * (T * D) + e_off
                handles[("x", ci, b)] = pltpu.async_copy(
                    x_hbm.at[pl.ds(x_off, CE)], xb.at[p, b], sx.at[p, b])

        load_chunk(0)
        for ci in range(n_chunks):
            p = ci % 2
            if ci + 1 < n_chunks:
                if ci >= 1:
                    for b in range(B):
                        handles[("s", ci - 1, b)].wait()
                load_chunk(ci + 1)
            handles[("e", ci)].wait()
            for b in range(B):
                handles[("x", ci, b)].wait()

                def add_body(i, _):
                    base = i * (16 * UNROLL)
                    vals = [eb[p, pl.ds(base + u * 16, 16)]
                            for u in range(UNROLL)]
                    for u in range(UNROLL):
                        plsc.addupdate(
                            xb.at[p, b, pl.ds(base + u * 16, 16)], vals[u])
                    return 0

                lax.fori_loop(0, n_vec // UNROLL, add_body, 0)
                x_off = b * (T * D) + t0 + ci * CE
                handles[("s", ci, b)] = pltpu.async_copy(
                    xb.at[p, b], out_hbm.at[pl.ds(x_off, CE)], st.at[p, b])
        for b in range(B):
            handles[("s", n_chunks - 2, b)].wait()
            handles[("s", n_chunks - 1, b)].wait()

    out = k(x_flat, emb_flat)
    return out.reshape(B, T, D)


# speedup vs baseline: 5.2286x; 3.1498x over previous
import functools
import jax
import jax.numpy as jnp
from jax import lax
from jax.experimental import pallas as pl
from jax.experimental.pallas import tpu as pltpu, tpu_sc as plsc

C_ROWS = 8
UNROLL = 8


def kernel(x, emb_weight):
    B, T, D = x.shape
    NW = 32
    rows_w = T // NW
    n_chunks = rows_w // C_ROWS
    n_grp = (C_ROWS * D) // (16 * UNROLL)
    grp_per_row = D // (16 * UNROLL)

    x2 = x.reshape(B * T, D)
    emb2 = emb_weight[:T]
    mesh = plsc.VectorSubcoreMesh(core_axis_name="c", subcore_axis_name="s")

    @functools.partial(
        pl.kernel,
        mesh=mesh,
        out_type=jax.ShapeDtypeStruct((B * T, D), jnp.float32),
        scratch_types=[
            pltpu.VMEM((2, B, C_ROWS, D), jnp.float32),
            pltpu.VMEM((2, C_ROWS, D), jnp.float32),
            pltpu.SemaphoreType.DMA((2, B)),
            pltpu.SemaphoreType.DMA((2,)),
            pltpu.SemaphoreType.DMA((2, B)),
        ],
        compiler_params=pltpu.CompilerParams(use_tc_tiling_on_sc=True),
    )
    def k(x_hbm, emb_hbm, out_hbm, xb, eb, sx, se, st):
        wid = lax.axis_index("s") * 2 + lax.axis_index("c")
        t0 = wid * rows_w

        handles = {}

        def load_chunk(ci):
            p = ci % 2
            r0 = t0 + ci * C_ROWS
            handles[("e", ci)] = pltpu.async_copy(
                emb_hbm.at[pl.ds(r0, C_ROWS)], eb.at[p], se.at[p])
            for b in range(B):
                handles[("x", ci, b)] = pltpu.async_copy(
                    x_hbm.at[pl.ds(b * T + r0, C_ROWS)], xb.at[p, b],
                    sx.at[p, b])

        load_chunk(0)
        for ci in range(n_chunks):
            p = ci % 2
            if ci + 1 < n_chunks:
                if ci >= 1:
                    for b in range(B):
                        handles[("s", ci - 1, b)].wait()
                load_chunk(ci + 1)
            handles[("e", ci)].wait()
            for b in range(B):
                handles[("x", ci, b)].wait()

                def add_body(g, _):
                    r = g // grp_per_row
                    c0 = (g % grp_per_row) * (16 * UNROLL)
                    vals = [eb[p, r, pl.ds(c0 + u * 16, 16)]
                            for u in range(UNROLL)]
                    for u in range(UNROLL):
                        plsc.addupdate(
                            xb.at[p, b, r, pl.ds(c0 + u * 16, 16)], vals[u])
                    return 0

                lax.fori_loop(0, n_grp, add_body, 0)
                r0 = t0 + ci * C_ROWS
                handles[("s", ci, b)] = pltpu.async_copy(
                    xb.at[p, b], out_hbm.at[pl.ds(b * T + r0, C_ROWS)],
                    st.at[p, b])
        for b in range(B):
            handles[("s", n_chunks - 2, b)].wait()
            handles[("s", n_chunks - 1, b)].wait()

    out = k(x2, emb2)
    return out.reshape(B, T, D)
